# Initial kernel scaffold; baseline (speedup 1.0000x reference)
import jax
import jax.numpy as jnp
from jax.experimental import pallas as pl


def kernel(x, reverse):
    raise NotImplementedError("write your pallas kernel here")


# speedup vs baseline: 1.1896x; 1.1896x over previous
import functools

import jax
import jax.numpy as jnp
from jax import lax
from jax.experimental import pallas as pl
from jax.experimental.pallas import tpu as pltpu
from jax.experimental.pallas import tpu_sc as plsc

L = 16
RADIX_BITS = 8
RADIX = 1 << RADIX_BITS
MASK = RADIX - 1
SIGN = -0x80000000


def _i32(x):
    return plsc.bitcast(x, jnp.int32)


def _f32(x):
    return plsc.bitcast(x, jnp.float32)


@functools.partial(jax.jit, static_argnums=(2, 3))
def _sc_sort(x_flat, rev_vec, rows, n):
    info = plsc.get_sparse_core_info()
    nc, ns = info.num_cores, info.num_subcores
    nw = nc * ns
    rpw = rows // nw
    seg = n // L

    mesh = plsc.VectorSubcoreMesh(core_axis_name="c", subcore_axis_name="s")

    @functools.partial(
        pl.kernel,
        out_type=(
            jax.ShapeDtypeStruct((rows * n,), jnp.float32),
            jax.ShapeDtypeStruct((rows * n,), jnp.int32),
        ),
        mesh=mesh,
        compiler_params=pltpu.CompilerParams(needs_layout_passes=False),
        scratch_types=[
            pltpu.VMEM((n,), jnp.float32),
            pltpu.VMEM((n,), jnp.int32),
            pltpu.VMEM((n,), jnp.float32),
            pltpu.VMEM((RADIX * L,), jnp.int32),
            pltpu.VMEM((L,), jnp.int32),
        ],
    )
    def sortk(x_hbm, rev_hbm, vals_hbm, idx_hbm, keyu, buf_a, buf_b, hist, revv):
        wid = lax.axis_index("s") * nc + lax.axis_index("c")
        pltpu.sync_copy(rev_hbm, revv)
        xm = jnp.where(revv[...] != 0, jnp.full((L,), -1, jnp.int32),
                       jnp.zeros((L,), jnp.int32))
        lane = lax.iota(jnp.int32, L)
        pbase = lane * seg
        ones = jnp.ones((L,), jnp.int32)

        def radix_pass(src, src_is_f32, dst, dst_is_f32, shift):
            def zero_body(i, _):
                hist[pl.ds(i * L, L)] = jnp.zeros((L,), jnp.int32)
                return 0

            lax.fori_loop(0, RADIX, zero_body, 0, unroll=4)

            def get_iv(p):
                if src is None:
                    return p
                if src_is_f32:
                    return _i32(plsc.load_gather(src, [p]))
                return plsc.load_gather(src, [p])

            def digit_of(iv):
                g = _i32(plsc.load_gather(keyu, [iv]))
                return (g >> shift) & MASK

            def hist_body(t, _):
                iv = get_iv(pbase + t)
                d = digit_of(iv)
                plsc.addupdate_scatter(hist, [d * L + lane], ones)
                return 0

            lax.fori_loop(0, seg, hist_body, 0, unroll=4)

            def scan_body(i, carry):
                h = hist[pl.ds(i * L, L)]
                c = plsc.cumsum(h)
                hist[pl.ds(i * L, L)] = c - h + carry
                return carry + jnp.sum(h)

            lax.fori_loop(0, RADIX, scan_body, jnp.int32(0), unroll=4)

            def perm_body(t, _):
                iv = get_iv(pbase + t)
                hidx = digit_of(iv) * L + lane
                off = plsc.load_gather(hist, [hidx])
                plsc.store_scatter(dst, [off], _f32(iv) if dst_is_f32 else iv)
                plsc.addupdate_scatter(hist, [hidx], ones)
                return 0

            lax.fori_loop(0, seg, perm_body, 0)

        for r in range(rpw):
            row = wid * rpw + r
            base = row * n
            pltpu.sync_copy(x_hbm.at[pl.ds(base, n)], keyu)

            def tf_body(i, _):
                b = _i32(keyu[pl.ds(i * L, L)])
                u = (b ^ ((b >> 31) | SIGN)) ^ xm
                keyu[pl.ds(i * L, L)] = _f32(u)
                return 0

            lax.fori_loop(0, seg, tf_body, 0, unroll=4)

            radix_pass(None, False, buf_b, True, 0)
            radix_pass(buf_b, True, buf_a, False, 8)
            radix_pass(buf_a, False, buf_b, True, 16)
            radix_pass(buf_b, True, buf_a, False, 24)

            pltpu.sync_copy(buf_a, idx_hbm.at[pl.ds(base, n)])

            def val_body(i, _):
                iv = buf_a[pl.ds(i * L, L)]
                v = _i32(plsc.load_gather(keyu, [iv])) ^ xm
                b = v ^ (~(v >> 31) | SIGN)
                buf_b[pl.ds(i * L, L)] = _f32(b)
                return 0

            lax.fori_loop(0, seg, val_body, 0, unroll=4)

            pltpu.sync_copy(buf_b, vals_hbm.at[pl.ds(base, n)])

    return sortk(x_flat, rev_vec)


def kernel(x, reverse):
    rows, n = x.shape
    rev_vec = jnp.full((L,), reverse, dtype=jnp.int32)
    vals, idx = _sc_sort(x.reshape(-1), rev_vec, rows, n)
    return vals.reshape(rows, n), idx.reshape(rows, n)


# speedup vs baseline: 1.8753x; 1.5764x over previous
import functools

import jax
import jax.numpy as jnp
from jax import lax
from jax.experimental import pallas as pl
from jax.experimental.pallas import tpu as pltpu
from jax.experimental.pallas import tpu_sc as plsc

L = 16
C = 4
PASS_BITS = (11, 11, 10)
PASS_SHIFTS = (0, 11, 22)
NBINS_MAX = 1 << max(PASS_BITS)
SIGN = -0x80000000


def _i32(x):
    return plsc.bitcast(x, jnp.int32)


def _f32(x):
    return plsc.bitcast(x, jnp.float32)


@functools.partial(jax.jit, static_argnums=(2, 3))
def _sc_sort(x_flat, rev_vec, rows, n):
    info = plsc.get_sparse_core_info()
    nc, ns = info.num_cores, info.num_subcores
    nw = nc * ns
    rpw = rows // nw
    seg = n // L
    tb = seg // C

    mesh = plsc.VectorSubcoreMesh(core_axis_name="c", subcore_axis_name="s")

    @functools.partial(
        pl.kernel,
        out_type=(
            jax.ShapeDtypeStruct((rows * n,), jnp.float32),
            jax.ShapeDtypeStruct((rows * n,), jnp.int32),
        ),
        mesh=mesh,
        compiler_params=pltpu.CompilerParams(needs_layout_passes=False),
        scratch_types=[
            pltpu.VMEM((n,), jnp.float32),
            pltpu.VMEM((n,), jnp.int32),
            pltpu.VMEM((n,), jnp.float32),
            pltpu.VMEM((NBINS_MAX,), jnp.int32),
            pltpu.VMEM((NBINS_MAX,), jnp.int32),
            pltpu.VMEM((NBINS_MAX,), jnp.int32),
            pltpu.VMEM((NBINS_MAX,), jnp.int32),
            pltpu.VMEM((L,), jnp.int32),
        ],
    )
    def sortk(x_hbm, rev_hbm, vals_hbm, idx_hbm,
              keyu, buf_a, buf_b, h0, h1, h2, h3, revv):
        hists = (h0, h1, h2, h3)
        wid = lax.axis_index("s") * nc + lax.axis_index("c")
        pltpu.sync_copy(rev_hbm, revv)
        xm = jnp.where(revv[...] != 0, jnp.full((L,), -1, jnp.int32),
                       jnp.zeros((L,), jnp.int32))
        lane = lax.iota(jnp.int32, L)

        def run_pass(pidx, src, src_is_f32, dst, dst_is_f32):
            shift = PASS_SHIFTS[pidx]
            nb = 1 << PASS_BITS[pidx]
            dmask = nb - 1

            def zero_body(i, _):
                z = jnp.zeros((L,), jnp.int32)
                for h in hists:
                    h[pl.ds(i * L, L)] = z
                return 0

            lax.fori_loop(0, nb // L, zero_body, 0, unroll=4)

            def load_iv_key(c, t):
                base = (c * tb + t) * L
                if src is None:
                    iv = base + lane
                    key = _i32(keyu[pl.ds(base, L)])
                else:
                    iv = src[pl.ds(base, L)]
                    if src_is_f32:
                        iv = _i32(iv)
                    key = _i32(plsc.load_gather(keyu, [iv]))
                return iv, key

            def hist_body(t, _):
                for c, h in enumerate(hists):
                    _, key = load_iv_key(c, t)
                    d = (key >> shift) & dmask
                    cnt, last = plsc.scan_count(d)
                    plsc.addupdate_scatter(h, [d], cnt, mask=last)
                return 0

            lax.fori_loop(0, tb, hist_body, 0, unroll=2)

            def scan_body(i, carry):
                hv = [h[pl.ds(i * L, L)] for h in hists]
                tot = hv[0] + hv[1] + hv[2] + hv[3]
                cum = plsc.cumsum(tot)
                b = cum - tot + carry
                for c, h in enumerate(hists):
                    h[pl.ds(i * L, L)] = b
                    if c < C - 1:
                        b = b + hv[c]
                return carry + cum[L - 1]

            lax.fori_loop(0, nb // L, scan_body, jnp.int32(0), unroll=2)

            def perm_body(t, _):
                for c, h in enumerate(hists):
                    iv, key = load_iv_key(c, t)
                    d = (key >> shift) & dmask
                    cnt, last = plsc.scan_count(d)
                    base = plsc.load_gather(h, [d])
                    rank = base + cnt - 1
                    plsc.store_scatter(dst, [rank],
                                       _f32(iv) if dst_is_f32 else iv)
                    plsc.addupdate_scatter(h, [d], cnt, mask=last)
                return 0

            lax.fori_loop(0, tb, perm_body, 0, unroll=2)

        for r in range(rpw):
            row = wid * rpw + r
            hbase = row * n
            pltpu.sync_copy(x_hbm.at[pl.ds(hbase, n)], keyu)

            def tf_body(i, _):
                b = _i32(keyu[pl.ds(i * L, L)])
                u = (b ^ ((b >> 31) | SIGN)) ^ xm
                keyu[pl.ds(i * L, L)] = _f32(u)
                return 0

            lax.fori_loop(0, seg, tf_body, 0, unroll=4)

            run_pass(0, None, False, buf_a, False)
            run_pass(1, buf_a, False, buf_b, True)
            run_pass(2, buf_b, True, buf_a, False)

            pltpu.sync_copy(buf_a, idx_hbm.at[pl.ds(hbase, n)])

            def val_body(i, _):
                iv = buf_a[pl.ds(i * L, L)]
                v = _i32(plsc.load_gather(keyu, [iv])) ^ xm
                b = v ^ (~(v >> 31) | SIGN)
                buf_b[pl.ds(i * L, L)] = _f32(b)
                return 0

            lax.fori_loop(0, seg, val_body, 0, unroll=4)

            pltpu.sync_copy(buf_b, vals_hbm.at[pl.ds(hbase, n)])

    return sortk(x_flat, rev_vec)


def kernel(x, reverse):
    rows, n = x.shape
    rev_vec = jnp.full((L,), reverse, dtype=jnp.int32)
    vals, idx = _sc_sort(x.reshape(-1), rev_vec, rows, n)
    return vals.reshape(rows, n), idx.reshape(rows, n)


# speedup vs baseline: 3.2722x; 1.7449x over previous
import functools

import jax
import jax.numpy as jnp
from jax import lax
from jax.experimental import pallas as pl
from jax.experimental.pallas import tpu as pltpu
from jax.experimental.pallas import tpu_sc as plsc

L = 16
C = 4
PASS_BITS = (11, 11, 10)
PASS_SHIFTS = (0, 11, 22)
NBINS_MAX = 1 << max(PASS_BITS)
SIGN = -0x80000000


def _i32(x):
    return plsc.bitcast(x, jnp.int32)


def _f32(x):
    return plsc.bitcast(x, jnp.float32)


@functools.partial(jax.jit, static_argnums=(2, 3))
def _sc_sort(x_flat, rev_vec, rows, n):
    info = plsc.get_sparse_core_info()
    nc, ns = info.num_cores, info.num_subcores
    nw = nc * ns
    rpw = rows // nw
    seg = n // L
    tb = seg // C

    mesh = plsc.VectorSubcoreMesh(core_axis_name="c", subcore_axis_name="s")

    @functools.partial(
        pl.kernel,
        out_type=(
            jax.ShapeDtypeStruct((rows * n,), jnp.float32),
            jax.ShapeDtypeStruct((rows * n,), jnp.int32),
        ),
        mesh=mesh,
        compiler_params=pltpu.CompilerParams(needs_layout_passes=False),
        scratch_types=[
            pltpu.VMEM((n,), jnp.float32),
            pltpu.VMEM((n,), jnp.int32),
            pltpu.VMEM((n,), jnp.float32),
            pltpu.VMEM((NBINS_MAX,), jnp.int32),
            pltpu.VMEM((NBINS_MAX,), jnp.int32),
            pltpu.VMEM((NBINS_MAX,), jnp.int32),
            pltpu.VMEM((NBINS_MAX,), jnp.int32),
            pltpu.VMEM((L,), jnp.int32),
        ],
    )
    def sortk(x_hbm, rev_hbm, vals_hbm, idx_hbm,
              keyu, buf_a, buf_b, h0, h1, h2, h3, revv):
        hists = (h0, h1, h2, h3)
        wid = lax.axis_index("s") * nc + lax.axis_index("c")
        pltpu.sync_copy(rev_hbm, revv)
        xm = jnp.where(revv[...] != 0, jnp.full((L,), -1, jnp.int32),
                       jnp.zeros((L,), jnp.int32))
        lane = lax.iota(jnp.int32, L)

        def run_pass(pidx, src, src_is_f32, dst, dst_is_f32):
            shift = PASS_SHIFTS[pidx]
            nb = 1 << PASS_BITS[pidx]
            dmask = nb - 1

            @plsc.parallel_loop(0, nb // L, unroll=4)
            def zero_body(i):
                z = jnp.zeros((L,), jnp.int32)
                for h in hists:
                    h[pl.ds(i * L, L)] = z

            def load_iv_key(c, t):
                base = (c * tb + t) * L
                if src is None:
                    iv = base + lane
                    key = _i32(keyu[pl.ds(base, L)])
                else:
                    iv = src[pl.ds(base, L)]
                    if src_is_f32:
                        iv = _i32(iv)
                    key = _i32(plsc.load_gather(keyu, [iv]))
                return iv, key

            @plsc.parallel_loop(0, tb, unroll=2)
            def hist_body(t):
                for c, h in enumerate(hists):
                    _, key = load_iv_key(c, t)
                    d = (key >> shift) & dmask
                    cnt, last = plsc.scan_count(d)
                    plsc.addupdate_scatter(h, [d], cnt, mask=last)

            def scan_body(i, carry):
                hv = [h[pl.ds(i * L, L)] for h in hists]
                tot = hv[0] + hv[1] + hv[2] + hv[3]
                cum = plsc.cumsum(tot)
                b = cum - tot + carry
                for c, h in enumerate(hists):
                    h[pl.ds(i * L, L)] = b
                    if c < C - 1:
                        b = b + hv[c]
                return carry + cum[L - 1]

            lax.fori_loop(0, nb // L, scan_body, jnp.int32(0), unroll=2)

            def perm_body(t, _):
                for c, h in enumerate(hists):
                    iv, key = load_iv_key(c, t)
                    d = (key >> shift) & dmask
                    cnt, last = plsc.scan_count(d)
                    base = plsc.load_gather(h, [d])
                    rank = base + cnt - 1
                    plsc.store_scatter(dst, [rank],
                                       _f32(iv) if dst_is_f32 else iv)
                    plsc.addupdate_scatter(h, [d], cnt, mask=last)
                return 0

            lax.fori_loop(0, tb, perm_body, 0, unroll=2)

        def row_body(r, _):
            row = wid * rpw + r
            hbase = row * n
            pltpu.sync_copy(x_hbm.at[pl.ds(hbase, n)], keyu)

            @plsc.parallel_loop(0, seg, unroll=4)
            def tf_body(i):
                b = _i32(keyu[pl.ds(i * L, L)])
                u = (b ^ ((b >> 31) | SIGN)) ^ xm
                keyu[pl.ds(i * L, L)] = _f32(u)

            run_pass(0, None, False, buf_a, False)
            run_pass(1, buf_a, False, buf_b, True)
            run_pass(2, buf_b, True, buf_a, False)

            pltpu.sync_copy(buf_a, idx_hbm.at[pl.ds(hbase, n)])

            @plsc.parallel_loop(0, seg, unroll=4)
            def val_body(i):
                iv = buf_a[pl.ds(i * L, L)]
                v = _i32(plsc.load_gather(keyu, [iv])) ^ xm
                b = v ^ (~(v >> 31) | SIGN)
                buf_b[pl.ds(i * L, L)] = _f32(b)

            pltpu.sync_copy(buf_b, vals_hbm.at[pl.ds(hbase, n)])
            return 0

        lax.fori_loop(0, rpw, row_body, 0)

    return sortk(x_flat, rev_vec)


def kernel(x, reverse):
    rows, n = x.shape
    rev_vec = jnp.full((L,), reverse, dtype=jnp.int32)
    vals, idx = _sc_sort(x.reshape(-1), rev_vec, rows, n)
    return vals.reshape(rows, n), idx.reshape(rows, n)
